# baseline (speedup 1.0000x reference)
import functools

import jax
import jax.numpy as jnp
from jax import lax
from jax.experimental import pallas as pl
from jax.experimental.pallas import tpu as pltpu
from jax.experimental.pallas import tpu_sc as plsc

N = 10000
NP = 10240
E = 320000
CH = 128
EP = 327680
NCHT = EP // CH
TBIG = 104
TSML = 56
BIGC = 0
RPT = NP // 16
NBUF = 2
TD = NCHT // 32

_MESH_KW = dict(core_axis_name="c", subcore_axis_name="s")


def _make_deg():
    @functools.partial(
        pl.kernel,
        out_type=jax.ShapeDtypeStruct((2, NP, 16), jnp.float32),
        mesh=plsc.VectorSubcoreMesh(**_MESH_KW),
        compiler_params=pltpu.CompilerParams(use_tc_tiling_on_sc=False),
        scratch_types=[
            pltpu.VMEM((TD, CH), jnp.int32),
            pltpu.VMEM((CH, 16), jnp.float32),
            pltpu.VMEM_SHARED((NP, 16), jnp.float32),
        ],
    )
    def deg_kernel(dst_hbm, ones_hbm, zeros_hbm, out_hbm, dst_v, ones_v, acc):
        c = lax.axis_index("c")
        s = lax.axis_index("s")
        wid = c * 16 + s
        pltpu.sync_copy(dst_hbm.at[pl.ds(wid * TD, TD)], dst_v)
        pltpu.sync_copy(ones_hbm, ones_v)
        for r in range(RPT // 128):
            pltpu.sync_copy(zeros_hbm, acc.at[pl.ds(s * RPT + r * 128, 128)])
        plsc.subcore_barrier()

        def body(j, carry):
            pltpu.sync_copy(ones_v, acc.at[dst_v.at[j]], add=True)
            return carry

        lax.fori_loop(0, TD, body, 0)
        plsc.subcore_barrier()
        for r in range(RPT // 128):
            rr = s * RPT + r * 128
            pltpu.sync_copy(acc.at[pl.ds(rr, 128)], out_hbm.at[c, pl.ds(rr, 128)])

    return deg_kernel


def _make_agg(d_feat, nbuf):
    @functools.partial(
        pl.kernel,
        out_type=jax.ShapeDtypeStruct((2, NP, d_feat), jnp.float32),
        mesh=plsc.VectorSubcoreMesh(**_MESH_KW),
        compiler_params=pltpu.CompilerParams(use_tc_tiling_on_sc=False),
        scratch_types=[
            pltpu.VMEM((TBIG // 2, CH), jnp.int32),
            pltpu.VMEM((TBIG // 2, CH), jnp.int32),
            [pltpu.VMEM((CH, d_feat), jnp.float32) for _ in range(nbuf)],
            pltpu.VMEM_SHARED((NP, d_feat), jnp.float32),
            [pltpu.SemaphoreType.DMA for _ in range(nbuf)],
        ],
    )
    def agg_kernel(hs_hbm, src_hbm, dst_hbm, zeros_hbm, out_hbm,
                   src_v, dst_v, gbufs, acc, gsems):
        c = lax.axis_index("c")
        s = lax.axis_index("s")
        for r in range(RPT // 128):
            pltpu.sync_copy(zeros_hbm, acc.at[pl.ds(s * RPT + r * 128, 128)])

        def run(t_chunks, base):
            pltpu.sync_copy(src_hbm.at[pl.ds(base, t_chunks)],
                            src_v.at[pl.ds(0, t_chunks)])
            pltpu.sync_copy(dst_hbm.at[pl.ds(base, t_chunks)],
                            dst_v.at[pl.ds(0, t_chunks)])
            for b in range(nbuf):
                pltpu.async_copy(hs_hbm.at[src_v.at[b]], gbufs[b], gsems[b])
            ngroup = t_chunks // nbuf

            def group(p, carry):
                for b in range(nbuf):
                    j = p * nbuf + b
                    pltpu.make_async_copy(hs_hbm.at[src_v.at[j]], gbufs[b],
                                          gsems[b]).wait()
                    pltpu.sync_copy(gbufs[b], acc.at[dst_v.at[j]], add=True)
                    pltpu.async_copy(hs_hbm.at[src_v.at[j + nbuf]], gbufs[b],
                                     gsems[b])
                return carry

            lax.fori_loop(0, ngroup - 1, group, 0)
            for b in range(nbuf):
                j = (ngroup - 1) * nbuf + b
                pltpu.make_async_copy(hs_hbm.at[src_v.at[j]], gbufs[b],
                                      gsems[b]).wait()
                pltpu.sync_copy(gbufs[b], acc.at[dst_v.at[j]], add=True)

        @pl.when(c == BIGC)
        def _():
            run(TBIG // 2, s * TBIG)
            run(TBIG // 2, s * TBIG + TBIG // 2)

        @pl.when(c == 1 - BIGC)
        def _():
            run(TSML // 2, 16 * TBIG + s * TSML)
            run(TSML // 2, 16 * TBIG + s * TSML + TSML // 2)

        plsc.subcore_barrier()
        for r in range(RPT // 128):
            rr = s * RPT + r * 128
            pltpu.sync_copy(acc.at[pl.ds(rr, 128)], out_hbm.at[c, pl.ds(rr, 128)])

    return agg_kernel


_deg_kernel = _make_deg()
_agg128 = _make_agg(128, NBUF)
_agg64 = _make_agg(64, NBUF)


def _tc_pre(x_ref, w1t_ref, deg_ref, out_ref):
    d = deg_ref[...]
    s = lax.rsqrt(1.0 + d[0, :, 0:1] + d[1, :, 0:1])
    h = jnp.dot(x_ref[...], w1t_ref[...], preferred_element_type=jnp.float32)
    out_ref[...] = h * s


def _tc_mid(acc_ref, hs1_ref, deg_ref, b1_ref, w2t_ref, out_ref):
    d = deg_ref[...]
    s = lax.rsqrt(1.0 + d[0, :, 0:1] + d[1, :, 0:1])
    a = acc_ref[...]
    pre = (a[0] + a[1] + hs1_ref[...]) * s + b1_ref[...]
    h1 = jnp.maximum(pre, 0.0)
    out_ref[...] = jnp.dot(h1, w2t_ref[...], preferred_element_type=jnp.float32) * s


def _tc_post(acc_ref, hs2_ref, deg_ref, b2_ref, out_ref):
    d = deg_ref[...]
    s = lax.rsqrt(1.0 + d[0, :, 0:1] + d[1, :, 0:1])
    a = acc_ref[...]
    out_ref[...] = (a[0] + a[1] + hs2_ref[...]) * s + b2_ref[...]


def kernel(x, edge_index, W1, b1, W2, b2):
    src = edge_index[0]
    dst = edge_index[1]
    pad = jnp.full((EP - E,), N, dtype=jnp.int32)
    src2 = jnp.concatenate([src, pad]).reshape(NCHT, CH)
    dst2 = jnp.concatenate([dst, pad]).reshape(NCHT, CH)
    xp = jnp.pad(x, ((0, NP - N), (0, 0)))
    w1t = W1.T
    w2t = W2.T
    o16 = jnp.ones((CH, 16), jnp.float32)
    z16 = jnp.zeros((128, 16), jnp.float32)
    z128 = jnp.zeros((128, 128), jnp.float32)
    z64 = jnp.zeros((128, 64), jnp.float32)

    deg16 = _deg_kernel(dst2, o16, z16)

    hs1 = pl.pallas_call(
        _tc_pre,
        out_shape=jax.ShapeDtypeStruct((NP, 128), jnp.float32),
    )(xp, w1t, deg16)

    acc1 = _agg128(hs1, src2, dst2, z128)

    hs2 = pl.pallas_call(
        _tc_mid,
        out_shape=jax.ShapeDtypeStruct((NP, 64), jnp.float32),
    )(acc1, hs1, deg16, b1.reshape(1, 128), w2t)

    acc2 = _agg64(hs2, src2, dst2, z64)

    outp = pl.pallas_call(
        _tc_post,
        out_shape=jax.ShapeDtypeStruct((NP, 64), jnp.float32),
    )(acc2, hs2, deg16, b2.reshape(1, 64))

    return outp[:N]


# speedup vs baseline: 1.0193x; 1.0193x over previous
import functools

import jax
import jax.numpy as jnp
from jax import lax
from jax.experimental import pallas as pl
from jax.experimental.pallas import tpu as pltpu
from jax.experimental.pallas import tpu_sc as plsc

N = 10000
NP = 10240
E = 320000
CH = 64
EP = 327680
NCHT = EP // CH
TBIG = 240
TSML = 80
BIGC = 0
RPT = NP // 16
NBUF = 2
TD = NCHT // 32

_MESH_KW = dict(core_axis_name="c", subcore_axis_name="s")


def _make_deg():
    @functools.partial(
        pl.kernel,
        out_type=jax.ShapeDtypeStruct((2, NP, 16), jnp.float32),
        mesh=plsc.VectorSubcoreMesh(**_MESH_KW),
        compiler_params=pltpu.CompilerParams(use_tc_tiling_on_sc=False),
        scratch_types=[
            pltpu.VMEM((TD, CH), jnp.int32),
            pltpu.VMEM((CH, 16), jnp.float32),
            pltpu.VMEM_SHARED((NP, 16), jnp.float32),
        ],
    )
    def deg_kernel(dst_hbm, ones_hbm, zeros_hbm, out_hbm, dst_v, ones_v, acc):
        c = lax.axis_index("c")
        s = lax.axis_index("s")
        wid = c * 16 + s
        pltpu.sync_copy(dst_hbm.at[pl.ds(wid * TD, TD)], dst_v)
        pltpu.sync_copy(ones_hbm, ones_v)
        for r in range(RPT // 128):
            pltpu.sync_copy(zeros_hbm, acc.at[pl.ds(s * RPT + r * 128, 128)])
        plsc.subcore_barrier()

        def body(j, carry):
            pltpu.sync_copy(ones_v, acc.at[dst_v.at[j]], add=True)
            return carry

        lax.fori_loop(0, TD, body, 0)
        plsc.subcore_barrier()
        for r in range(RPT // 128):
            rr = s * RPT + r * 128
            pltpu.sync_copy(acc.at[pl.ds(rr, 128)], out_hbm.at[c, pl.ds(rr, 128)])

    return deg_kernel


def _make_agg(d_feat, nbuf):
    @functools.partial(
        pl.kernel,
        out_type=jax.ShapeDtypeStruct((2, NP, d_feat), jnp.float32),
        mesh=plsc.VectorSubcoreMesh(**_MESH_KW),
        compiler_params=pltpu.CompilerParams(use_tc_tiling_on_sc=False),
        scratch_types=[
            pltpu.VMEM((TBIG, CH), jnp.int32),
            pltpu.VMEM((TBIG, CH), jnp.int32),
            [pltpu.VMEM((CH, d_feat), jnp.float32) for _ in range(nbuf)],
            pltpu.VMEM_SHARED((NP, d_feat), jnp.float32),
            [pltpu.SemaphoreType.DMA for _ in range(nbuf)],
        ],
    )
    def agg_kernel(hs_hbm, src_hbm, dst_hbm, zeros_hbm, out_hbm,
                   src_v, dst_v, gbufs, acc, gsems):
        c = lax.axis_index("c")
        s = lax.axis_index("s")
        for r in range(RPT // 128):
            pltpu.sync_copy(zeros_hbm, acc.at[pl.ds(s * RPT + r * 128, 128)])

        def run(t_chunks, base):
            pltpu.sync_copy(src_hbm.at[pl.ds(base, t_chunks)],
                            src_v.at[pl.ds(0, t_chunks)])
            pltpu.sync_copy(dst_hbm.at[pl.ds(base, t_chunks)],
                            dst_v.at[pl.ds(0, t_chunks)])
            for b in range(nbuf):
                pltpu.async_copy(hs_hbm.at[src_v.at[b]], gbufs[b], gsems[b])
            ngroup = t_chunks // nbuf

            def group(p, carry):
                for b in range(nbuf):
                    j = p * nbuf + b
                    pltpu.make_async_copy(hs_hbm.at[src_v.at[j]], gbufs[b],
                                          gsems[b]).wait()
                    pltpu.sync_copy(gbufs[b], acc.at[dst_v.at[j]], add=True)
                    pltpu.async_copy(hs_hbm.at[src_v.at[j + nbuf]], gbufs[b],
                                     gsems[b])
                return carry

            lax.fori_loop(0, ngroup - 1, group, 0)
            for b in range(nbuf):
                j = (ngroup - 1) * nbuf + b
                pltpu.make_async_copy(hs_hbm.at[src_v.at[j]], gbufs[b],
                                      gsems[b]).wait()
                pltpu.sync_copy(gbufs[b], acc.at[dst_v.at[j]], add=True)

        @pl.when(c == BIGC)
        def _():
            run(TBIG, s * TBIG)

        @pl.when(c == 1 - BIGC)
        def _():
            run(TSML, 16 * TBIG + s * TSML)

        plsc.subcore_barrier()
        for r in range(RPT // 128):
            rr = s * RPT + r * 128
            pltpu.sync_copy(acc.at[pl.ds(rr, 128)], out_hbm.at[c, pl.ds(rr, 128)])

    return agg_kernel


_deg_kernel = _make_deg()
_agg128 = _make_agg(128, NBUF)
_agg64 = _make_agg(64, NBUF)


def _tc_pre(x_ref, w1t_ref, deg_ref, out_ref):
    d = deg_ref[...]
    s = lax.rsqrt(1.0 + d[0, :, 0:1] + d[1, :, 0:1])
    h = jnp.dot(x_ref[...], w1t_ref[...], preferred_element_type=jnp.float32)
    out_ref[...] = h * s


def _tc_mid(acc_ref, hs1_ref, deg_ref, b1_ref, w2t_ref, out_ref):
    d = deg_ref[...]
    s = lax.rsqrt(1.0 + d[0, :, 0:1] + d[1, :, 0:1])
    a = acc_ref[...]
    pre = (a[0] + a[1] + hs1_ref[...]) * s + b1_ref[...]
    h1 = jnp.maximum(pre, 0.0)
    out_ref[...] = jnp.dot(h1, w2t_ref[...], preferred_element_type=jnp.float32) * s


def _tc_post(acc_ref, hs2_ref, deg_ref, b2_ref, out_ref):
    d = deg_ref[...]
    s = lax.rsqrt(1.0 + d[0, :, 0:1] + d[1, :, 0:1])
    a = acc_ref[...]
    out_ref[...] = (a[0] + a[1] + hs2_ref[...]) * s + b2_ref[...]


def kernel(x, edge_index, W1, b1, W2, b2):
    src = edge_index[0]
    dst = edge_index[1]
    pad = jnp.full((EP - E,), N, dtype=jnp.int32)
    src2 = jnp.concatenate([src, pad]).reshape(NCHT, CH)
    dst2 = jnp.concatenate([dst, pad]).reshape(NCHT, CH)
    xp = jnp.pad(x, ((0, NP - N), (0, 0)))
    w1t = W1.T
    w2t = W2.T
    o16 = jnp.ones((CH, 16), jnp.float32)
    z16 = jnp.zeros((128, 16), jnp.float32)
    z128 = jnp.zeros((128, 128), jnp.float32)
    z64 = jnp.zeros((128, 64), jnp.float32)

    deg16 = _deg_kernel(dst2, o16, z16)

    hs1 = pl.pallas_call(
        _tc_pre,
        out_shape=jax.ShapeDtypeStruct((NP, 128), jnp.float32),
    )(xp, w1t, deg16)

    acc1 = _agg128(hs1, src2, dst2, z128)

    hs2 = pl.pallas_call(
        _tc_mid,
        out_shape=jax.ShapeDtypeStruct((NP, 64), jnp.float32),
    )(acc1, hs1, deg16, b1.reshape(1, 128), w2t)

    acc2 = _agg64(hs2, src2, dst2, z64)

    outp = pl.pallas_call(
        _tc_post,
        out_shape=jax.ShapeDtypeStruct((NP, 64), jnp.float32),
    )(acc2, hs2, deg16, b2.reshape(1, 64))

    return outp[:N]
